# Initial kernel scaffold; baseline (speedup 1.0000x reference)
import jax
import jax.numpy as jnp
from jax.experimental import pallas as pl


def kernel(x_cls, x_patch, params):
    raise NotImplementedError("write your pallas kernel here")


# speedup vs baseline: 5.2757x; 5.2757x over previous
import functools

import jax
import jax.numpy as jnp
from jax.experimental import pallas as pl

C = 384
CLSN = 150
NH = 4
HD = C // NH
HP = 32
WP = 32
N = HP * WP
KNN = 9
G = NH * CLSN
EPS = 1e-5
SCALE = HD ** -0.5


def _ln_rows(x, g, b):
    m = jnp.mean(x, axis=1, keepdims=True)
    v = jnp.mean((x - m) ** 2, axis=1, keepdims=True)
    return (x - m) * jax.lax.rsqrt(v + EPS) * g + b


def _lng_block(x, w, b):
    m = jnp.mean(x)
    v = jnp.mean((x - m) ** 2)
    return (x - m) * jax.lax.rsqrt(v + EPS) * w + b


def _dot(a, b):
    return jax.lax.dot_general(a, b, (((1,), (0,)), ((), ())),
                               preferred_element_type=jnp.float32)


def _dot_tb(a, b):
    return jax.lax.dot_general(a, b, (((1,), (1,)), ((), ())),
                               preferred_element_type=jnp.float32)


def _gelu(x):
    return jax.nn.gelu(x, approximate=True)


def _k1_body(xc_ref, xp_ref, ncls_g, ncls_b, qwt, qb, kvw, kvb, nxw, nxb,
             projwt, projb, w9, ppb, ppg, ppbb, out_cls_ref, xp2_ref):
    xc = xc_ref[0]
    xp = xp_ref[0]
    xl = _ln_rows(xc, ncls_g[...], ncls_b[...])
    q = _dot(xl, qwt[...]) + qb[...]
    xn = _lng_block(xp, nxw[...], nxb[...])
    kv = _dot(kvw[...], xn) + kvb[...]
    outs = []
    for h in range(NH):
        qh = q[:, h * HD:(h + 1) * HD]
        kh = kv[h * HD:(h + 1) * HD, :]
        vh = kv[C + h * HD:C + (h + 1) * HD, :]
        lg = _dot(qh, kh) * SCALE
        lg = lg - jnp.max(lg, axis=1, keepdims=True)
        e = jnp.exp(lg)
        p = e / jnp.sum(e, axis=1, keepdims=True)
        outs.append(_dot_tb(p, vh))
    oc = jnp.concatenate(outs, axis=1)
    out_cls = xc + _dot(oc, projwt[...]) + projb[...]
    out_cls_ref[0] = out_cls

    op = _dot(out_cls, xp)
    col = jax.lax.broadcasted_iota(jnp.int32, (1, N), 1) % WP
    row = jax.lax.broadcasted_iota(jnp.int32, (1, N), 1) // WP
    acc = jnp.zeros((C, N), jnp.float32)
    for ky in range(3):
        for kx in range(3):
            off = (ky - 1) * WP + (kx - 1)
            if off > 0:
                sh = jnp.concatenate(
                    [op[:, off:], jnp.zeros((CLSN, off), jnp.float32)], axis=1)
            elif off < 0:
                sh = jnp.concatenate(
                    [jnp.zeros((CLSN, -off), jnp.float32), op[:, :N + off]],
                    axis=1)
            else:
                sh = op
            mask = ((col + (kx - 1) >= 0) & (col + (kx - 1) < WP) &
                    (row + (ky - 1) >= 0) & (row + (ky - 1) < HP))
            sh = jnp.where(mask, sh, 0.0)
            acc = acc + _dot(w9[3 * ky + kx], sh)
    op2 = (acc + ppb[...]) * ppg[...] + ppbb[...]
    xp2_ref[0] = xp + _gelu(op2)


def _k2a_body(cls_ref, xp2_ref, ncls_g, ncls_b, qw, qb, kvwt, kvb, nxw, nxb,
              attn_ref, vv_ref):
    xp2 = xp2_ref[0]
    xn = _lng_block(xp2, nxw[...], nxb[...])
    q = _dot(qw[...], xn) + qb[...]
    clsn = _ln_rows(cls_ref[0], ncls_g[...], ncls_b[...])
    kv = _dot(clsn, kvwt[...]) + kvb[...]
    kk = kv[:, :C]
    vv_ref[0] = kv[:, C:]
    blocks = []
    for h in range(NH):
        kh = kk[:, h * HD:(h + 1) * HD]
        qh = q[h * HD:(h + 1) * HD, :]
        blocks.append(_dot(kh, qh) * SCALE)
    attn_ref[0] = jnp.concatenate(blocks, axis=0)


def _k2b_body(x_ref, fc1w, fc1b, fc1g, fc1bb, wf, wm, nnb, nng, nnbb,
              fc2w, fc2b, fc2g, fc2bb, out_ref):
    x = x_ref[0]
    x1 = _dot(fc1w[...], x) + fc1b[...]
    x1 = x1 * fc1g[...] + fc1bb[...]

    f = x1.T
    gram = _dot_tb(f, f)
    sq_col = jnp.sum(f * f, axis=1, keepdims=True)
    sq_row = jnp.sum(x1 * x1, axis=0, keepdims=True)
    dist = sq_col - 2.0 * gram + sq_row

    iota = jax.lax.broadcasted_iota(jnp.int32, (N, N), 1)
    maxn = jnp.full((G, N), -jnp.inf, jnp.float32)
    for _ in range(KNN):
        vmin = jnp.min(dist, axis=1, keepdims=True)
        eq = dist <= vmin
        idx = jnp.min(jnp.where(eq, iota, jnp.int32(2 ** 30)), axis=1,
                      keepdims=True)
        oh = (iota == idx)
        gat = _dot_tb(x1, oh.astype(jnp.float32))
        maxn = jnp.maximum(maxn, gat)
        dist = jnp.where(oh, jnp.inf, dist)

    mrel = maxn - x1
    ys = []
    for g in range(NH):
        xg = x1[g * CLSN:(g + 1) * CLSN, :]
        mg = mrel[g * CLSN:(g + 1) * CLSN, :]
        ys.append(_dot(wf[g], xg) + _dot(wm[g], mg))
    y = jnp.concatenate(ys, axis=0) + nnb[...]
    y = _gelu(y * nng[...] + nnbb[...])
    out = _dot(fc2w[...], y) + fc2b[...]
    out_ref[0] = out * fc2g[...] + fc2bb[...] + x


def _k2c_body(gout_ref, vv_ref, cls_ref, xp2_ref, projw, projb,
              normg, normb, m1wt, m1b, m2wt, m2b,
              cls_out_ref, patch_out_ref):
    gout = gout_ref[0]
    vv = vv_ref[0]
    vvt = vv.T
    outs = []
    for h in range(NH):
        blk = gout[h * CLSN:(h + 1) * CLSN, :]
        blk = blk - jnp.max(blk, axis=0, keepdims=True)
        e = jnp.exp(blk)
        p = e / jnp.sum(e, axis=0, keepdims=True)
        vh = vvt[h * HD:(h + 1) * HD, :]
        outs.append(_dot(vh, p))
    o = jnp.concatenate(outs, axis=0)
    patch_out_ref[0] = xp2_ref[0] + _dot(projw[...], o) + projb[...]

    xc = cls_ref[0]
    hl = _ln_rows(xc, normg[...], normb[...])
    h1 = _gelu(_dot(hl, m1wt[...]) + m1b[...])
    h2 = _dot(h1, m2wt[...]) + m2b[...]
    cls_out_ref[0] = xc + h2


def _bspec(shape):
    nz = (0,) * len(shape)
    return pl.BlockSpec(shape, lambda b, _z=nz: _z)


def _bspecB(shape):
    nz = (0,) * len(shape)
    return pl.BlockSpec((1,) + shape, lambda b, _z=nz: (b,) + _z)


def _call(body, batch, ins, in_shapes, out_shapes):
    in_specs = [(_bspecB(s) if bt else _bspec(s)) for (_, bt), s in
                zip(ins, in_shapes)]
    out_specs = [_bspecB(s) for s in out_shapes]
    out_shape = [jax.ShapeDtypeStruct((batch,) + s, jnp.float32)
                 for s in out_shapes]
    return pl.pallas_call(
        body, grid=(batch,), in_specs=in_specs, out_specs=out_specs,
        out_shape=out_shape,
    )(*[a for a, _ in ins])


def kernel(x_cls, x_patch, params):
    batch = x_cls.shape[0]
    f32 = jnp.float32
    p1 = params['s2f']
    p2 = params['f2s']
    pg = p2['grapher']
    xp = x_patch.reshape(batch, C, N)

    r2 = lambda a: a.reshape(-1, 1).astype(f32)
    r1 = lambda a: a.reshape(1, -1).astype(f32)

    w9 = p1['pp_w'].transpose(2, 3, 0, 1).reshape(9, C, CLSN)
    k1_ins = [
        (x_cls, True), (xp, True),
        (r1(p1['ncls_g']), False), (r1(p1['ncls_b']), False),
        (p1['q_w'].T, False), (r1(p1['q_b']), False),
        (p1['kv_w'], False), (r2(p1['kv_b']), False),
        (r2(p1['nx_w']), False), (r2(p1['nx_b']), False),
        (p1['proj_w'].T, False), (r1(p1['proj_b']), False),
        (w9, False), (r2(p1['pp_b']), False),
        (r2(p1['pp_bn_g']), False), (r2(p1['pp_bn_b']), False),
    ]
    k1_shapes = [(CLSN, C), (C, N), (1, C), (1, C), (C, C), (1, C),
                 (2 * C, C), (2 * C, 1), (C, 1), (C, 1), (C, C), (1, C),
                 (9, C, CLSN), (C, 1), (C, 1), (C, 1)]
    out_cls, xp2 = _call(_k1_body, batch, k1_ins, k1_shapes,
                         [(CLSN, C), (C, N)])

    k2a_ins = [
        (out_cls, True), (xp2, True),
        (r1(p2['ncls_g']), False), (r1(p2['ncls_b']), False),
        (p2['q_w'], False), (r2(p2['q_b']), False),
        (p2['kv_w'].T, False), (r1(p2['kv_b']), False),
        (r2(p2['nx_w']), False), (r2(p2['nx_b']), False),
    ]
    k2a_shapes = [(CLSN, C), (C, N), (1, C), (1, C), (C, C), (C, 1),
                  (C, 2 * C), (1, 2 * C), (C, 1), (C, 1)]
    attn_pre, vv = _call(_k2a_body, batch, k2a_ins, k2a_shapes,
                         [(G, N), (CLSN, C)])

    wf = pg['nn_w'][:, :, 0::2]
    wm = pg['nn_w'][:, :, 1::2]
    gpg = 2 * G // NH
    k2b_ins = [
        (attn_pre, True),
        (pg['fc1_w'], False), (r2(pg['fc1_b']), False),
        (r2(pg['fc1_bn_g']), False), (r2(pg['fc1_bn_b']), False),
        (wf, False), (wm, False),
        (r2(pg['nn_b']), False), (r2(pg['nn_bn_g']), False),
        (r2(pg['nn_bn_b']), False),
        (pg['fc2_w'], False), (r2(pg['fc2_b']), False),
        (r2(pg['fc2_bn_g']), False), (r2(pg['fc2_bn_b']), False),
    ]
    k2b_shapes = [(G, N), (G, G), (G, 1), (G, 1), (G, 1),
                  (NH, gpg, CLSN), (NH, gpg, CLSN),
                  (2 * G, 1), (2 * G, 1), (2 * G, 1),
                  (G, 2 * G), (G, 1), (G, 1), (G, 1)]
    (gout,) = _call(_k2b_body, batch, k2b_ins, k2b_shapes, [(G, N)])

    k2c_ins = [
        (gout, True), (vv, True), (out_cls, True), (xp2, True),
        (p2['proj_w'], False), (r2(p2['proj_b']), False),
        (r1(params['norm_g']), False), (r1(params['norm_b']), False),
        (params['mlp_fc1_w'].T, False), (r1(params['mlp_fc1_b']), False),
        (params['mlp_fc2_w'].T, False), (r1(params['mlp_fc2_b']), False),
    ]
    k2c_shapes = [(G, N), (CLSN, C), (CLSN, C), (C, N), (C, C), (C, 1),
                  (1, C), (1, C), (C, 4 * C), (1, 4 * C), (4 * C, C),
                  (1, C)]
    cls_out, patch_out = _call(_k2c_body, batch, k2c_ins, k2c_shapes,
                               [(CLSN, C), (C, N)])
    return cls_out, patch_out.reshape(batch, C, HP, WP)


# speedup vs baseline: 5.4608x; 1.0351x over previous
import functools

import jax
import jax.numpy as jnp
from jax.experimental import pallas as pl
from jax.experimental.pallas import tpu as pltpu
from jax.experimental.pallas import tpu_sc as plsc

C = 384
CLSN = 150
NH = 4
HD = C // NH
HP = 32
WP = 32
N = HP * WP
KNN = 9
G = NH * CLSN
EPS = 1e-5
SCALE = HD ** -0.5
DPAD = 640
NWORK = 32


def _ln_rows(x, g, b):
    m = jnp.mean(x, axis=1, keepdims=True)
    v = jnp.mean((x - m) ** 2, axis=1, keepdims=True)
    return (x - m) * jax.lax.rsqrt(v + EPS) * g + b


def _lng_block(x, w, b):
    m = jnp.mean(x)
    v = jnp.mean((x - m) ** 2)
    return (x - m) * jax.lax.rsqrt(v + EPS) * w + b


def _dot(a, b):
    return jax.lax.dot_general(a, b, (((1,), (0,)), ((), ())),
                               preferred_element_type=jnp.float32)


def _dot_tb(a, b):
    return jax.lax.dot_general(a, b, (((1,), (1,)), ((), ())),
                               preferred_element_type=jnp.float32)


def _gelu(x):
    return jax.nn.gelu(x, approximate=True)


def _k1_body(xc_ref, xp_ref, ncls_g, ncls_b, qwt, qb, kvw, kvb, nxw, nxb,
             projwt, projb, w9, ppb, ppg, ppbb, out_cls_ref, xp2_ref):
    xc = xc_ref[0]
    xp = xp_ref[0]
    xl = _ln_rows(xc, ncls_g[...], ncls_b[...])
    q = _dot(xl, qwt[...]) + qb[...]
    xn = _lng_block(xp, nxw[...], nxb[...])
    kv = _dot(kvw[...], xn) + kvb[...]
    outs = []
    for h in range(NH):
        qh = q[:, h * HD:(h + 1) * HD]
        kh = kv[h * HD:(h + 1) * HD, :]
        vh = kv[C + h * HD:C + (h + 1) * HD, :]
        lg = _dot(qh, kh) * SCALE
        lg = lg - jnp.max(lg, axis=1, keepdims=True)
        e = jnp.exp(lg)
        p = e / jnp.sum(e, axis=1, keepdims=True)
        outs.append(_dot_tb(p, vh))
    oc = jnp.concatenate(outs, axis=1)
    out_cls = xc + _dot(oc, projwt[...]) + projb[...]
    out_cls_ref[0] = out_cls

    op = _dot(out_cls, xp)
    col = jax.lax.broadcasted_iota(jnp.int32, (1, N), 1) % WP
    row = jax.lax.broadcasted_iota(jnp.int32, (1, N), 1) // WP
    acc = jnp.zeros((C, N), jnp.float32)
    for ky in range(3):
        for kx in range(3):
            off = (ky - 1) * WP + (kx - 1)
            if off > 0:
                sh = jnp.concatenate(
                    [op[:, off:], jnp.zeros((CLSN, off), jnp.float32)], axis=1)
            elif off < 0:
                sh = jnp.concatenate(
                    [jnp.zeros((CLSN, -off), jnp.float32), op[:, :N + off]],
                    axis=1)
            else:
                sh = op
            mask = ((col + (kx - 1) >= 0) & (col + (kx - 1) < WP) &
                    (row + (ky - 1) >= 0) & (row + (ky - 1) < HP))
            sh = jnp.where(mask, sh, 0.0)
            acc = acc + _dot(w9[3 * ky + kx], sh)
    op2 = (acc + ppb[...]) * ppg[...] + ppbb[...]
    xp2_ref[0] = xp + _gelu(op2)


def _k2a_body(cls_ref, xp2_ref, ncls_g, ncls_b, qw, qb, kvwt, kvb, nxw, nxb,
              attn_ref, vv_ref):
    xp2 = xp2_ref[0]
    xn = _lng_block(xp2, nxw[...], nxb[...])
    q = _dot(qw[...], xn) + qb[...]
    clsn = _ln_rows(cls_ref[0], ncls_g[...], ncls_b[...])
    kv = _dot(clsn, kvwt[...]) + kvb[...]
    kk = kv[:, :C]
    vv_ref[0] = kv[:, C:]
    blocks = []
    for h in range(NH):
        kh = kk[:, h * HD:(h + 1) * HD]
        qh = q[h * HD:(h + 1) * HD, :]
        blocks.append(_dot(kh, qh) * SCALE)
    attn_ref[0] = jnp.concatenate(blocks, axis=0)


def _k2b1_body(x_ref, fc1w, fc1b, fc1g, fc1bb, x1_ref, fpad_ref, idx_ref):
    x = x_ref[0]
    x1 = _dot(fc1w[...], x) + fc1b[...]
    x1 = x1 * fc1g[...] + fc1bb[...]
    x1_ref[0] = x1

    f = x1.T
    fpad_ref[0, :, :G] = f
    fpad_ref[0, :, G:] = jnp.zeros((N, DPAD - G), jnp.float32)

    gram = _dot_tb(f, f)
    sq_col = jnp.sum(f * f, axis=1, keepdims=True)
    sq_row = jnp.sum(x1 * x1, axis=0, keepdims=True)
    dist = sq_col - 2.0 * gram + sq_row

    gbase = pl.program_id(0) * N
    iota = jax.lax.broadcasted_iota(jnp.int32, (N, N), 1)
    for k in range(KNN):
        vmin = jnp.min(dist, axis=1, keepdims=True)
        eq = dist <= vmin
        idx = jnp.min(jnp.where(eq, iota, jnp.int32(2 ** 30)), axis=1,
                      keepdims=True)
        idx_ref[0, :, k:k + 1] = idx + gbase
        if k < KNN - 1:
            dist = jnp.where(iota == idx, jnp.inf, dist)


def _sc_gather_max(table, idx):
    rows, d = table.shape
    npw = rows // NWORK
    mesh = plsc.VectorSubcoreMesh(core_axis_name="c", subcore_axis_name="s")

    @functools.partial(
        pl.kernel, mesh=mesh,
        out_type=jax.ShapeDtypeStruct((rows, d), jnp.float32),
        scratch_types=[
            pltpu.VMEM((npw,), jnp.int32),
            pltpu.VMEM((npw, d), jnp.float32),
            pltpu.VMEM((npw, d), jnp.float32),
            pltpu.SemaphoreType.DMA,
        ],
    )
    def run(table_hbm, idx_hbm, out_hbm, idxv, accv, rowv, sem):
        wid = jax.lax.axis_index("s") * 2 + jax.lax.axis_index("c")
        base = wid * npw
        pltpu.sync_copy(idx_hbm.at[0, pl.ds(base, npw)], idxv)
        pltpu.async_copy(table_hbm.at[idxv], accv, sem).wait()
        for k in range(1, KNN):
            pltpu.sync_copy(idx_hbm.at[k, pl.ds(base, npw)], idxv)
            pltpu.async_copy(table_hbm.at[idxv], rowv, sem).wait()

            def body(i, _):
                for j in range(d // 16):
                    sl = pl.ds(j * 16, 16)
                    accv[i, sl] = jnp.maximum(accv[i, sl], rowv[i, sl])
                return 0

            jax.lax.fori_loop(0, npw, body, 0)
        pltpu.sync_copy(accv, out_hbm.at[pl.ds(base, npw)])

    return run(table, idx)


def _k2b2_body(x_ref, x1_ref, mt_ref, wfm, wm, nnb, nng, nnbb,
               fc2w, fc2b, fc2g, fc2bb, out_ref):
    x = x_ref[0]
    x1 = x1_ref[0]
    mt = mt_ref[0]
    ys = []
    for g in range(NH):
        xg = x1[g * CLSN:(g + 1) * CLSN, :]
        mtg = mt[:, g * CLSN:(g + 1) * CLSN]
        ys.append(_dot(wfm[g], xg) + _dot_tb(wm[g], mtg))
    y = jnp.concatenate(ys, axis=0) + nnb[...]
    y = _gelu(y * nng[...] + nnbb[...])
    out = _dot(fc2w[...], y) + fc2b[...]
    out_ref[0] = out * fc2g[...] + fc2bb[...] + x


def _k2c_body(gout_ref, vv_ref, cls_ref, xp2_ref, projw, projb,
              normg, normb, m1wt, m1b, m2wt, m2b,
              cls_out_ref, patch_out_ref):
    gout = gout_ref[0]
    vv = vv_ref[0]
    vvt = vv.T
    outs = []
    for h in range(NH):
        blk = gout[h * CLSN:(h + 1) * CLSN, :]
        blk = blk - jnp.max(blk, axis=0, keepdims=True)
        e = jnp.exp(blk)
        p = e / jnp.sum(e, axis=0, keepdims=True)
        vh = vvt[h * HD:(h + 1) * HD, :]
        outs.append(_dot(vh, p))
    o = jnp.concatenate(outs, axis=0)
    patch_out_ref[0] = xp2_ref[0] + _dot(projw[...], o) + projb[...]

    xc = cls_ref[0]
    hl = _ln_rows(xc, normg[...], normb[...])
    h1 = _gelu(_dot(hl, m1wt[...]) + m1b[...])
    h2 = _dot(h1, m2wt[...]) + m2b[...]
    cls_out_ref[0] = xc + h2


def _bspec(shape):
    nz = (0,) * len(shape)
    return pl.BlockSpec(shape, lambda b, _z=nz: _z)


def _bspecB(shape):
    nz = (0,) * len(shape)
    return pl.BlockSpec((1,) + shape, lambda b, _z=nz: (b,) + _z)


def _call(body, batch, ins, in_shapes, out_shapes, out_dtypes=None):
    in_specs = [(_bspecB(s) if bt else _bspec(s)) for (_, bt), s in
                zip(ins, in_shapes)]
    out_specs = [_bspecB(s) for s in out_shapes]
    if out_dtypes is None:
        out_dtypes = [jnp.float32] * len(out_shapes)
    out_shape = [jax.ShapeDtypeStruct((batch,) + s, dt)
                 for s, dt in zip(out_shapes, out_dtypes)]
    return pl.pallas_call(
        body, grid=(batch,), in_specs=in_specs, out_specs=out_specs,
        out_shape=out_shape,
    )(*[a for a, _ in ins])


def kernel(x_cls, x_patch, params):
    batch = x_cls.shape[0]
    f32 = jnp.float32
    p1 = params['s2f']
    p2 = params['f2s']
    pg = p2['grapher']
    xp = x_patch.reshape(batch, C, N)

    r2 = lambda a: a.reshape(-1, 1).astype(f32)
    r1 = lambda a: a.reshape(1, -1).astype(f32)

    w9 = p1['pp_w'].transpose(2, 3, 0, 1).reshape(9, C, CLSN)
    k1_ins = [
        (x_cls, True), (xp, True),
        (r1(p1['ncls_g']), False), (r1(p1['ncls_b']), False),
        (p1['q_w'].T, False), (r1(p1['q_b']), False),
        (p1['kv_w'], False), (r2(p1['kv_b']), False),
        (r2(p1['nx_w']), False), (r2(p1['nx_b']), False),
        (p1['proj_w'].T, False), (r1(p1['proj_b']), False),
        (w9, False), (r2(p1['pp_b']), False),
        (r2(p1['pp_bn_g']), False), (r2(p1['pp_bn_b']), False),
    ]
    k1_shapes = [(CLSN, C), (C, N), (1, C), (1, C), (C, C), (1, C),
                 (2 * C, C), (2 * C, 1), (C, 1), (C, 1), (C, C), (1, C),
                 (9, C, CLSN), (C, 1), (C, 1), (C, 1)]
    out_cls, xp2 = _call(_k1_body, batch, k1_ins, k1_shapes,
                         [(CLSN, C), (C, N)])

    k2a_ins = [
        (out_cls, True), (xp2, True),
        (r1(p2['ncls_g']), False), (r1(p2['ncls_b']), False),
        (p2['q_w'], False), (r2(p2['q_b']), False),
        (p2['kv_w'].T, False), (r1(p2['kv_b']), False),
        (r2(p2['nx_w']), False), (r2(p2['nx_b']), False),
    ]
    k2a_shapes = [(CLSN, C), (C, N), (1, C), (1, C), (C, C), (C, 1),
                  (C, 2 * C), (1, 2 * C), (C, 1), (C, 1)]
    attn_pre, vv = _call(_k2a_body, batch, k2a_ins, k2a_shapes,
                         [(G, N), (CLSN, C)])

    k2b1_ins = [
        (attn_pre, True),
        (pg['fc1_w'], False), (r2(pg['fc1_b']), False),
        (r2(pg['fc1_bn_g']), False), (r2(pg['fc1_bn_b']), False),
    ]
    k2b1_shapes = [(G, N), (G, G), (G, 1), (G, 1), (G, 1)]
    x1b, fpad, idx = _call(_k2b1_body, batch, k2b1_ins, k2b1_shapes,
                           [(G, N), (N, DPAD), (N, KNN)],
                           [jnp.float32, jnp.float32, jnp.int32])

    table = fpad.reshape(batch * N, DPAD)
    idx_sc = idx.transpose(2, 0, 1).reshape(KNN, batch * N)
    maxnt = _sc_gather_max(table, idx_sc).reshape(batch, N, DPAD)

    wf = pg['nn_w'][:, :, 0::2]
    wm = pg['nn_w'][:, :, 1::2]
    wfm = wf - wm
    gpg = 2 * G // NH
    k2b2_ins = [
        (attn_pre, True), (x1b, True), (maxnt, True),
        (wfm, False), (wm, False),
        (r2(pg['nn_b']), False), (r2(pg['nn_bn_g']), False),
        (r2(pg['nn_bn_b']), False),
        (pg['fc2_w'], False), (r2(pg['fc2_b']), False),
        (r2(pg['fc2_bn_g']), False), (r2(pg['fc2_bn_b']), False),
    ]
    k2b2_shapes = [(G, N), (G, N), (N, DPAD),
                   (NH, gpg, CLSN), (NH, gpg, CLSN),
                   (2 * G, 1), (2 * G, 1), (2 * G, 1),
                   (G, 2 * G), (G, 1), (G, 1), (G, 1)]
    (gout,) = _call(_k2b2_body, batch, k2b2_ins, k2b2_shapes, [(G, N)])

    k2c_ins = [
        (gout, True), (vv, True), (out_cls, True), (xp2, True),
        (p2['proj_w'], False), (r2(p2['proj_b']), False),
        (r1(params['norm_g']), False), (r1(params['norm_b']), False),
        (params['mlp_fc1_w'].T, False), (r1(params['mlp_fc1_b']), False),
        (params['mlp_fc2_w'].T, False), (r1(params['mlp_fc2_b']), False),
    ]
    k2c_shapes = [(G, N), (CLSN, C), (CLSN, C), (C, N), (C, C), (C, 1),
                  (1, C), (1, C), (C, 4 * C), (1, 4 * C), (4 * C, C),
                  (1, C)]
    cls_out, patch_out = _call(_k2c_body, batch, k2c_ins, k2c_shapes,
                               [(CLSN, C), (C, N)])
    return cls_out, patch_out.reshape(batch, C, HP, WP)


# speedup vs baseline: 6.0602x; 1.1098x over previous
import functools

import jax
import jax.numpy as jnp
from jax.experimental import pallas as pl
from jax.experimental.pallas import tpu as pltpu
from jax.experimental.pallas import tpu_sc as plsc

C = 384
CLSN = 150
NH = 4
HD = C // NH
HP = 32
WP = 32
N = HP * WP
KNN = 9
G = NH * CLSN
EPS = 1e-5
SCALE = HD ** -0.5
DPAD = 640
NWORK = 32


def _ln_rows(x, g, b):
    m = jnp.mean(x, axis=1, keepdims=True)
    v = jnp.mean((x - m) ** 2, axis=1, keepdims=True)
    return (x - m) * jax.lax.rsqrt(v + EPS) * g + b


def _lng_block(x, w, b):
    m = jnp.mean(x)
    v = jnp.mean((x - m) ** 2)
    return (x - m) * jax.lax.rsqrt(v + EPS) * w + b


def _dot(a, b):
    return jax.lax.dot_general(a, b, (((1,), (0,)), ((), ())),
                               preferred_element_type=jnp.float32)


def _dot_tb(a, b):
    return jax.lax.dot_general(a, b, (((1,), (1,)), ((), ())),
                               preferred_element_type=jnp.float32)


def _gelu(x):
    return jax.nn.gelu(x, approximate=True)


def _ka_body(xc_ref, xp_ref, ncls_g, ncls_b, qw, qb, kvw, kvb, nxw, nxb,
             projw, projb, w9, ppb, ppg, ppbb,
             ncls_g2, ncls_b2, qw2, qb2, kvw2, kvb2, nxw2, nxb2,
             out_cls_ref, xp2_ref, attn_ref, vv_ref):
    xc = xc_ref[0]
    xp = xp_ref[0]
    xl = _ln_rows(xc, ncls_g[...], ncls_b[...])
    q = _dot_tb(xl, qw[...]) + qb[...]
    xn = _lng_block(xp, nxw[...], nxb[...])
    kv = _dot(kvw[...], xn) + kvb[...]
    outs = []
    for h in range(NH):
        qh = q[:, h * HD:(h + 1) * HD]
        kh = kv[h * HD:(h + 1) * HD, :]
        vh = kv[C + h * HD:C + (h + 1) * HD, :]
        lg = _dot(qh, kh) * SCALE
        lg = lg - jnp.max(lg, axis=1, keepdims=True)
        e = jnp.exp(lg)
        p = e / jnp.sum(e, axis=1, keepdims=True)
        outs.append(_dot_tb(p, vh))
    oc = jnp.concatenate(outs, axis=1)
    out_cls = xc + _dot_tb(oc, projw[...]) + projb[...]
    out_cls_ref[0] = out_cls

    op = _dot(out_cls, xp)
    col = jax.lax.broadcasted_iota(jnp.int32, (1, N), 1) % WP
    row = jax.lax.broadcasted_iota(jnp.int32, (1, N), 1) // WP
    acc = jnp.zeros((C, N), jnp.float32)
    for ky in range(3):
        for kx in range(3):
            off = (ky - 1) * WP + (kx - 1)
            if off > 0:
                sh = jnp.concatenate(
                    [op[:, off:], jnp.zeros((CLSN, off), jnp.float32)], axis=1)
            elif off < 0:
                sh = jnp.concatenate(
                    [jnp.zeros((CLSN, -off), jnp.float32), op[:, :N + off]],
                    axis=1)
            else:
                sh = op
            mask = ((col + (kx - 1) >= 0) & (col + (kx - 1) < WP) &
                    (row + (ky - 1) >= 0) & (row + (ky - 1) < HP))
            sh = jnp.where(mask, sh, 0.0)
            acc = acc + _dot(w9[3 * ky + kx], sh)
    op2 = (acc + ppb[...]) * ppg[...] + ppbb[...]
    xp2 = xp + _gelu(op2)
    xp2_ref[0] = xp2

    xn2 = _lng_block(xp2, nxw2[...], nxb2[...])
    q2 = _dot(qw2[...], xn2) + qb2[...]
    clsn = _ln_rows(out_cls, ncls_g2[...], ncls_b2[...])
    kv2 = _dot_tb(clsn, kvw2[...]) + kvb2[...]
    kk = kv2[:, :C]
    vv_ref[0] = kv2[:, C:]
    blocks = []
    for h in range(NH):
        kh = kk[:, h * HD:(h + 1) * HD]
        qh = q2[h * HD:(h + 1) * HD, :]
        blocks.append(_dot(kh, qh) * SCALE)
    attn_ref[0] = jnp.concatenate(blocks, axis=0)


def _kb_body(x_ref, fc1w, fc1b, fc1g, fc1bb, x1_ref, fpad_ref, idx_ref):
    x = x_ref[0]
    x1 = _dot(fc1w[...], x) + fc1b[...]
    x1 = x1 * fc1g[...] + fc1bb[...]
    x1_ref[0] = x1

    f = x1.T
    fpad_ref[0, :, :G] = f
    fpad_ref[0, :, G:] = jnp.zeros((N, DPAD - G), jnp.float32)

    gram = _dot_tb(f, f)
    sq_row = jnp.sum(x1 * x1, axis=0, keepdims=True)
    dist = sq_row - 2.0 * gram

    gbase = pl.program_id(0) * N
    iotaf = jax.lax.broadcasted_iota(jnp.int32, (N, N), 1).astype(jnp.float32)
    for k in range(KNN):
        vmin = jnp.min(dist, axis=1, keepdims=True)
        idxf = jnp.min(jnp.where(dist <= vmin, iotaf, jnp.float32(2.0 * N)),
                       axis=1, keepdims=True)
        idx_ref[0, :, k:k + 1] = idxf.astype(jnp.int32) + gbase
        if k < KNN - 1:
            dist = jnp.where(iotaf == idxf, jnp.inf, dist)


def _sc_gather_max(table, idx):
    rows, d = table.shape
    npw = rows // NWORK
    mesh = plsc.VectorSubcoreMesh(core_axis_name="c", subcore_axis_name="s")

    @functools.partial(
        pl.kernel, mesh=mesh,
        out_type=jax.ShapeDtypeStruct((rows, d), jnp.float32),
        scratch_types=[
            pltpu.VMEM((npw,), jnp.int32),
            pltpu.VMEM((npw, d), jnp.float32),
            pltpu.VMEM((npw, d), jnp.float32),
            pltpu.SemaphoreType.DMA,
        ],
    )
    def run(table_hbm, idx_hbm, out_hbm, idxv, accv, rowv, sem):
        wid = jax.lax.axis_index("s") * 2 + jax.lax.axis_index("c")
        base = wid * npw
        pltpu.sync_copy(idx_hbm.at[0, pl.ds(base, npw)], idxv)
        pltpu.async_copy(table_hbm.at[idxv], accv, sem).wait()
        for k in range(1, KNN):
            pltpu.sync_copy(idx_hbm.at[k, pl.ds(base, npw)], idxv)
            pltpu.async_copy(table_hbm.at[idxv], rowv, sem).wait()

            def body(i, _):
                for j in range(d // 16):
                    sl = pl.ds(j * 16, 16)
                    accv[i, sl] = jnp.maximum(accv[i, sl], rowv[i, sl])
                return 0

            jax.lax.fori_loop(0, npw, body, 0)
        pltpu.sync_copy(accv, out_hbm.at[pl.ds(base, npw)])

    return run(table, idx)


def _kc_body(x_ref, x1_ref, mt_ref, vv_ref, cls_ref, xp2_ref,
             wfm, wm, nnb, nng, nnbb, fc2w, fc2b, fc2g, fc2bb,
             projw, projb, normg, normb, m1w, m1b, m2w, m2b,
             cls_out_ref, patch_out_ref):
    x = x_ref[0]
    x1 = x1_ref[0]
    mt = mt_ref[0]
    ys = []
    for g in range(NH):
        xg = x1[g * CLSN:(g + 1) * CLSN, :]
        mtg = mt[:, g * CLSN:(g + 1) * CLSN]
        ys.append(_dot(wfm[g], xg) + _dot_tb(wm[g], mtg))
    y = jnp.concatenate(ys, axis=0) + nnb[...]
    y = _gelu(y * nng[...] + nnbb[...])
    gout = _dot(fc2w[...], y) + fc2b[...]
    gout = gout * fc2g[...] + fc2bb[...] + x

    vv = vv_ref[0]
    vvt = vv.T
    outs = []
    for h in range(NH):
        blk = gout[h * CLSN:(h + 1) * CLSN, :]
        blk = blk - jnp.max(blk, axis=0, keepdims=True)
        e = jnp.exp(blk)
        p = e / jnp.sum(e, axis=0, keepdims=True)
        vh = vvt[h * HD:(h + 1) * HD, :]
        outs.append(_dot(vh, p))
    o = jnp.concatenate(outs, axis=0)
    patch_out_ref[0] = xp2_ref[0] + _dot(projw[...], o) + projb[...]

    xc = cls_ref[0]
    hl = _ln_rows(xc, normg[...], normb[...])
    h1 = _gelu(_dot_tb(hl, m1w[...]) + m1b[...])
    h2 = _dot_tb(h1, m2w[...]) + m2b[...]
    cls_out_ref[0] = xc + h2


def _bspec(shape):
    nz = (0,) * len(shape)
    return pl.BlockSpec(shape, lambda b, _z=nz: _z)


def _bspecB(shape):
    nz = (0,) * len(shape)
    return pl.BlockSpec((1,) + shape, lambda b, _z=nz: (b,) + _z)


def _call(body, batch, ins, in_shapes, out_shapes, out_dtypes=None):
    in_specs = [(_bspecB(s) if bt else _bspec(s)) for (_, bt), s in
                zip(ins, in_shapes)]
    out_specs = [_bspecB(s) for s in out_shapes]
    if out_dtypes is None:
        out_dtypes = [jnp.float32] * len(out_shapes)
    out_shape = [jax.ShapeDtypeStruct((batch,) + s, dt)
                 for s, dt in zip(out_shapes, out_dtypes)]
    return pl.pallas_call(
        body, grid=(batch,), in_specs=in_specs, out_specs=out_specs,
        out_shape=out_shape,
    )(*[a for a, _ in ins])


def kernel(x_cls, x_patch, params):
    batch = x_cls.shape[0]
    f32 = jnp.float32
    p1 = params['s2f']
    p2 = params['f2s']
    pg = p2['grapher']
    xp = x_patch.reshape(batch, C, N)

    r2 = lambda a: a.reshape(-1, 1).astype(f32)
    r1 = lambda a: a.reshape(1, -1).astype(f32)

    w9 = p1['pp_w'].transpose(2, 3, 0, 1).reshape(9, C, CLSN)
    ka_ins = [
        (x_cls, True), (xp, True),
        (r1(p1['ncls_g']), False), (r1(p1['ncls_b']), False),
        (p1['q_w'], False), (r1(p1['q_b']), False),
        (p1['kv_w'], False), (r2(p1['kv_b']), False),
        (r2(p1['nx_w']), False), (r2(p1['nx_b']), False),
        (p1['proj_w'], False), (r1(p1['proj_b']), False),
        (w9, False), (r2(p1['pp_b']), False),
        (r2(p1['pp_bn_g']), False), (r2(p1['pp_bn_b']), False),
        (r1(p2['ncls_g']), False), (r1(p2['ncls_b']), False),
        (p2['q_w'], False), (r2(p2['q_b']), False),
        (p2['kv_w'], False), (r1(p2['kv_b']), False),
        (r2(p2['nx_w']), False), (r2(p2['nx_b']), False),
    ]
    ka_shapes = [(CLSN, C), (C, N), (1, C), (1, C), (C, C), (1, C),
                 (2 * C, C), (2 * C, 1), (C, 1), (C, 1), (C, C), (1, C),
                 (9, C, CLSN), (C, 1), (C, 1), (C, 1),
                 (1, C), (1, C), (C, C), (C, 1), (2 * C, C), (1, 2 * C),
                 (C, 1), (C, 1)]
    out_cls, xp2, attn_pre, vv = _call(
        _ka_body, batch, ka_ins, ka_shapes,
        [(CLSN, C), (C, N), (G, N), (CLSN, C)])

    kb_ins = [
        (attn_pre, True),
        (pg['fc1_w'], False), (r2(pg['fc1_b']), False),
        (r2(pg['fc1_bn_g']), False), (r2(pg['fc1_bn_b']), False),
    ]
    kb_shapes = [(G, N), (G, G), (G, 1), (G, 1), (G, 1)]
    x1b, fpad, idx = _call(_kb_body, batch, kb_ins, kb_shapes,
                           [(G, N), (N, DPAD), (N, KNN)],
                           [jnp.float32, jnp.float32, jnp.int32])

    table = fpad.reshape(batch * N, DPAD)
    idx_sc = idx.transpose(2, 0, 1).reshape(KNN, batch * N)
    maxnt = _sc_gather_max(table, idx_sc).reshape(batch, N, DPAD)

    wf = pg['nn_w'][:, :, 0::2]
    wm = pg['nn_w'][:, :, 1::2]
    wfm = wf - wm
    gpg = 2 * G // NH
    kc_ins = [
        (attn_pre, True), (x1b, True), (maxnt, True),
        (vv, True), (out_cls, True), (xp2, True),
        (wfm, False), (wm, False),
        (r2(pg['nn_b']), False), (r2(pg['nn_bn_g']), False),
        (r2(pg['nn_bn_b']), False),
        (pg['fc2_w'], False), (r2(pg['fc2_b']), False),
        (r2(pg['fc2_bn_g']), False), (r2(pg['fc2_bn_b']), False),
        (p2['proj_w'], False), (r2(p2['proj_b']), False),
        (r1(params['norm_g']), False), (r1(params['norm_b']), False),
        (params['mlp_fc1_w'], False), (r1(params['mlp_fc1_b']), False),
        (params['mlp_fc2_w'], False), (r1(params['mlp_fc2_b']), False),
    ]
    kc_shapes = [(G, N), (G, N), (N, DPAD), (CLSN, C), (CLSN, C), (C, N),
                 (NH, gpg, CLSN), (NH, gpg, CLSN),
                 (2 * G, 1), (2 * G, 1), (2 * G, 1),
                 (G, 2 * G), (G, 1), (G, 1), (G, 1),
                 (C, C), (C, 1), (1, C), (1, C),
                 (4 * C, C), (1, 4 * C), (C, 4 * C), (1, C)]
    cls_out, patch_out = _call(_kc_body, batch, kc_ins, kc_shapes,
                               [(CLSN, C), (C, N)])
    return cls_out, patch_out.reshape(batch, C, HP, WP)


# speedup vs baseline: 6.0696x; 1.0016x over previous
import functools

import jax
import jax.numpy as jnp
from jax.experimental import pallas as pl
from jax.experimental.pallas import tpu as pltpu
from jax.experimental.pallas import tpu_sc as plsc

C = 384
CLSN = 150
NH = 4
HD = C // NH
HP = 32
WP = 32
N = HP * WP
KNN = 9
G = NH * CLSN
EPS = 1e-5
SCALE = HD ** -0.5
DPAD = 640
NWORK = 32


def _ln_rows(x, g, b):
    m = jnp.mean(x, axis=1, keepdims=True)
    v = jnp.mean((x - m) ** 2, axis=1, keepdims=True)
    return (x - m) * jax.lax.rsqrt(v + EPS) * g + b


def _lng_block(x, w, b):
    m = jnp.mean(x)
    v = jnp.mean((x - m) ** 2)
    return (x - m) * jax.lax.rsqrt(v + EPS) * w + b


def _dot(a, b):
    return jax.lax.dot_general(a, b, (((1,), (0,)), ((), ())),
                               preferred_element_type=jnp.float32)


def _dot_tb(a, b):
    return jax.lax.dot_general(a, b, (((1,), (1,)), ((), ())),
                               preferred_element_type=jnp.float32)


def _gelu(x):
    return jax.nn.gelu(x, approximate=True)


def _ka_body(xc_ref, xp_ref, ncls_g, ncls_b, qw, qb, kvw, kvb, nxw, nxb,
             projw, projb, w9, ppb, ppg, ppbb,
             ncls_g2, ncls_b2, qw2, qb2, kvw2, kvb2, nxw2, nxb2,
             out_cls_ref, xp2_ref, attn_ref, vv_ref):
    xc = xc_ref[0]
    xp = xp_ref[0]
    xl = _ln_rows(xc, ncls_g[...], ncls_b[...])
    q = _dot_tb(xl, qw[...]) + qb[...]
    xn = _lng_block(xp, nxw[...], nxb[...])
    kv = _dot(kvw[...], xn) + kvb[...]
    outs = []
    for h in range(NH):
        qh = q[:, h * HD:(h + 1) * HD]
        kh = kv[h * HD:(h + 1) * HD, :]
        vh = kv[C + h * HD:C + (h + 1) * HD, :]
        lg = _dot(qh, kh) * SCALE
        lg = lg - jnp.max(lg, axis=1, keepdims=True)
        e = jnp.exp(lg)
        p = e / jnp.sum(e, axis=1, keepdims=True)
        outs.append(_dot_tb(p, vh))
    oc = jnp.concatenate(outs, axis=1)
    out_cls = xc + _dot_tb(oc, projw[...]) + projb[...]
    out_cls_ref[0] = out_cls

    op = _dot(out_cls, xp)
    col = jax.lax.broadcasted_iota(jnp.int32, (1, N), 1) % WP
    row = jax.lax.broadcasted_iota(jnp.int32, (1, N), 1) // WP
    acc = jnp.zeros((C, N), jnp.float32)
    for ky in range(3):
        for kx in range(3):
            off = (ky - 1) * WP + (kx - 1)
            if off > 0:
                sh = jnp.concatenate(
                    [op[:, off:], jnp.zeros((CLSN, off), jnp.float32)], axis=1)
            elif off < 0:
                sh = jnp.concatenate(
                    [jnp.zeros((CLSN, -off), jnp.float32), op[:, :N + off]],
                    axis=1)
            else:
                sh = op
            mask = ((col + (kx - 1) >= 0) & (col + (kx - 1) < WP) &
                    (row + (ky - 1) >= 0) & (row + (ky - 1) < HP))
            sh = jnp.where(mask, sh, 0.0)
            acc = acc + _dot(w9[3 * ky + kx], sh)
    op2 = (acc + ppb[...]) * ppg[...] + ppbb[...]
    xp2 = xp + _gelu(op2)
    xp2_ref[0] = xp2

    xn2 = _lng_block(xp2, nxw2[...], nxb2[...])
    q2 = _dot(qw2[...], xn2) + qb2[...]
    clsn = _ln_rows(out_cls, ncls_g2[...], ncls_b2[...])
    kv2 = _dot_tb(clsn, kvw2[...]) + kvb2[...]
    kk = kv2[:, :C]
    vv_ref[0] = kv2[:, C:]
    blocks = []
    for h in range(NH):
        kh = kk[:, h * HD:(h + 1) * HD]
        qh = q2[h * HD:(h + 1) * HD, :]
        blocks.append(_dot(kh, qh) * SCALE)
    attn_ref[0] = jnp.concatenate(blocks, axis=0)


def _kb_body(x_ref, fc1w, fc1b, fc1g, fc1bb, x1_ref, fpad_ref, idx_ref):
    x = x_ref[0]
    x1 = _dot(fc1w[...], x) + fc1b[...]
    x1 = x1 * fc1g[...] + fc1bb[...]
    x1_ref[0] = x1

    f = x1.T
    fpad_ref[0, :, :G] = f
    fpad_ref[0, :, G:] = jnp.zeros((N, DPAD - G), jnp.float32)

    gram = _dot_tb(f, f)
    sq_row = jnp.sum(x1 * x1, axis=0, keepdims=True)
    dist = sq_row - 2.0 * gram

    gbase = pl.program_id(0) * N
    iotaf = jax.lax.broadcasted_iota(jnp.int32, (N, N), 1).astype(jnp.float32)
    for k in range(KNN):
        vmin = jnp.min(dist, axis=1, keepdims=True)
        idxf = jnp.min(jnp.where(dist <= vmin, iotaf, jnp.float32(2.0 * N)),
                       axis=1, keepdims=True)
        idx_ref[0, :, k:k + 1] = idxf.astype(jnp.int32) + gbase
        if k < KNN - 1:
            dist = jnp.where(iotaf == idxf, jnp.inf, dist)


def _sc_gather_max(table, idx):
    rows, d = table.shape
    npw = rows // NWORK
    mesh = plsc.VectorSubcoreMesh(core_axis_name="c", subcore_axis_name="s")

    nvr = (G + 15) // 16

    @functools.partial(
        pl.kernel, mesh=mesh,
        out_type=jax.ShapeDtypeStruct((rows, d), jnp.float32),
        scratch_types=[
            pltpu.VMEM((npw,), jnp.int32),
            pltpu.VMEM((npw,), jnp.int32),
            pltpu.VMEM((npw, d), jnp.float32),
            pltpu.VMEM((npw, d), jnp.float32),
            pltpu.VMEM((npw, d), jnp.float32),
            pltpu.SemaphoreType.DMA,
            pltpu.SemaphoreType.DMA,
            pltpu.SemaphoreType.DMA,
        ],
    )
    def run(table_hbm, idx_hbm, out_hbm, idxv0, idxv1, accv, row0, row1,
            sema, sem0, sem1):
        wid = jax.lax.axis_index("s") * 2 + jax.lax.axis_index("c")
        base = wid * npw
        idxs = (idxv0, idxv1)
        rows_ = (row0, row1)
        sems = (sem0, sem1)
        pltpu.sync_copy(idx_hbm.at[0, pl.ds(base, npw)], idxv0)
        pltpu.async_copy(table_hbm.at[idxv0], accv, sema).wait()
        pltpu.sync_copy(idx_hbm.at[1, pl.ds(base, npw)], idxv1)
        cps = {1: pltpu.async_copy(table_hbm.at[idxv1], row1, sem1)}
        for k in range(1, KNN):
            if k + 1 < KNN:
                nb = (k + 1) % 2
                pltpu.sync_copy(idx_hbm.at[k + 1, pl.ds(base, npw)],
                                idxs[nb])
                cps[k + 1] = pltpu.async_copy(table_hbm.at[idxs[nb]],
                                              rows_[nb], sems[nb])
            cps[k].wait()
            rowv = rows_[k % 2]

            def body(i, _):
                for j in range(nvr):
                    sl = pl.ds(j * 16, 16)
                    accv[i, sl] = jnp.maximum(accv[i, sl], rowv[i, sl])
                return 0

            jax.lax.fori_loop(0, npw, body, 0)
        pltpu.sync_copy(accv, out_hbm.at[pl.ds(base, npw)])

    return run(table, idx)


def _kc_body(x_ref, x1_ref, mt_ref, vv_ref, cls_ref, xp2_ref,
             wfm, wm, nnb, nng, nnbb, fc2w, fc2b, fc2g, fc2bb,
             projw, projb, normg, normb, m1w, m1b, m2w, m2b,
             cls_out_ref, patch_out_ref):
    x = x_ref[0]
    x1 = x1_ref[0]
    mt = mt_ref[0]
    ys = []
    for g in range(NH):
        xg = x1[g * CLSN:(g + 1) * CLSN, :]
        mtg = mt[:, g * CLSN:(g + 1) * CLSN]
        ys.append(_dot(wfm[g], xg) + _dot_tb(wm[g], mtg))
    y = jnp.concatenate(ys, axis=0) + nnb[...]
    y = _gelu(y * nng[...] + nnbb[...])
    gout = _dot(fc2w[...], y) + fc2b[...]
    gout = gout * fc2g[...] + fc2bb[...] + x

    vv = vv_ref[0]
    vvt = vv.T
    outs = []
    for h in range(NH):
        blk = gout[h * CLSN:(h + 1) * CLSN, :]
        blk = blk - jnp.max(blk, axis=0, keepdims=True)
        e = jnp.exp(blk)
        p = e / jnp.sum(e, axis=0, keepdims=True)
        vh = vvt[h * HD:(h + 1) * HD, :]
        outs.append(_dot(vh, p))
    o = jnp.concatenate(outs, axis=0)
    patch_out_ref[0] = xp2_ref[0] + _dot(projw[...], o) + projb[...]

    xc = cls_ref[0]
    hl = _ln_rows(xc, normg[...], normb[...])
    h1 = _gelu(_dot_tb(hl, m1w[...]) + m1b[...])
    h2 = _dot_tb(h1, m2w[...]) + m2b[...]
    cls_out_ref[0] = xc + h2


def _bspec(shape):
    nz = (0,) * len(shape)
    return pl.BlockSpec(shape, lambda b, _z=nz: _z)


def _bspecB(shape):
    nz = (0,) * len(shape)
    return pl.BlockSpec((1,) + shape, lambda b, _z=nz: (b,) + _z)


def _call(body, batch, ins, in_shapes, out_shapes, out_dtypes=None):
    in_specs = [(_bspecB(s) if bt else _bspec(s)) for (_, bt), s in
                zip(ins, in_shapes)]
    out_specs = [_bspecB(s) for s in out_shapes]
    if out_dtypes is None:
        out_dtypes = [jnp.float32] * len(out_shapes)
    out_shape = [jax.ShapeDtypeStruct((batch,) + s, dt)
                 for s, dt in zip(out_shapes, out_dtypes)]
    return pl.pallas_call(
        body, grid=(batch,), in_specs=in_specs, out_specs=out_specs,
        out_shape=out_shape,
    )(*[a for a, _ in ins])


def kernel(x_cls, x_patch, params):
    batch = x_cls.shape[0]
    f32 = jnp.float32
    p1 = params['s2f']
    p2 = params['f2s']
    pg = p2['grapher']
    xp = x_patch.reshape(batch, C, N)

    r2 = lambda a: a.reshape(-1, 1).astype(f32)
    r1 = lambda a: a.reshape(1, -1).astype(f32)

    w9 = p1['pp_w'].transpose(2, 3, 0, 1).reshape(9, C, CLSN)
    ka_ins = [
        (x_cls, True), (xp, True),
        (r1(p1['ncls_g']), False), (r1(p1['ncls_b']), False),
        (p1['q_w'], False), (r1(p1['q_b']), False),
        (p1['kv_w'], False), (r2(p1['kv_b']), False),
        (r2(p1['nx_w']), False), (r2(p1['nx_b']), False),
        (p1['proj_w'], False), (r1(p1['proj_b']), False),
        (w9, False), (r2(p1['pp_b']), False),
        (r2(p1['pp_bn_g']), False), (r2(p1['pp_bn_b']), False),
        (r1(p2['ncls_g']), False), (r1(p2['ncls_b']), False),
        (p2['q_w'], False), (r2(p2['q_b']), False),
        (p2['kv_w'], False), (r1(p2['kv_b']), False),
        (r2(p2['nx_w']), False), (r2(p2['nx_b']), False),
    ]
    ka_shapes = [(CLSN, C), (C, N), (1, C), (1, C), (C, C), (1, C),
                 (2 * C, C), (2 * C, 1), (C, 1), (C, 1), (C, C), (1, C),
                 (9, C, CLSN), (C, 1), (C, 1), (C, 1),
                 (1, C), (1, C), (C, C), (C, 1), (2 * C, C), (1, 2 * C),
                 (C, 1), (C, 1)]
    out_cls, xp2, attn_pre, vv = _call(
        _ka_body, batch, ka_ins, ka_shapes,
        [(CLSN, C), (C, N), (G, N), (CLSN, C)])

    kb_ins = [
        (attn_pre, True),
        (pg['fc1_w'], False), (r2(pg['fc1_b']), False),
        (r2(pg['fc1_bn_g']), False), (r2(pg['fc1_bn_b']), False),
    ]
    kb_shapes = [(G, N), (G, G), (G, 1), (G, 1), (G, 1)]
    x1b, fpad, idx = _call(_kb_body, batch, kb_ins, kb_shapes,
                           [(G, N), (N, DPAD), (N, KNN)],
                           [jnp.float32, jnp.float32, jnp.int32])

    table = fpad.reshape(batch * N, DPAD)
    idx_sc = idx.transpose(2, 0, 1).reshape(KNN, batch * N)
    maxnt = _sc_gather_max(table, idx_sc).reshape(batch, N, DPAD)

    wf = pg['nn_w'][:, :, 0::2]
    wm = pg['nn_w'][:, :, 1::2]
    wfm = wf - wm
    gpg = 2 * G // NH
    kc_ins = [
        (attn_pre, True), (x1b, True), (maxnt, True),
        (vv, True), (out_cls, True), (xp2, True),
        (wfm, False), (wm, False),
        (r2(pg['nn_b']), False), (r2(pg['nn_bn_g']), False),
        (r2(pg['nn_bn_b']), False),
        (pg['fc2_w'], False), (r2(pg['fc2_b']), False),
        (r2(pg['fc2_bn_g']), False), (r2(pg['fc2_bn_b']), False),
        (p2['proj_w'], False), (r2(p2['proj_b']), False),
        (r1(params['norm_g']), False), (r1(params['norm_b']), False),
        (params['mlp_fc1_w'], False), (r1(params['mlp_fc1_b']), False),
        (params['mlp_fc2_w'], False), (r1(params['mlp_fc2_b']), False),
    ]
    kc_shapes = [(G, N), (G, N), (N, DPAD), (CLSN, C), (CLSN, C), (C, N),
                 (NH, gpg, CLSN), (NH, gpg, CLSN),
                 (2 * G, 1), (2 * G, 1), (2 * G, 1),
                 (G, 2 * G), (G, 1), (G, 1), (G, 1),
                 (C, C), (C, 1), (1, C), (1, C),
                 (4 * C, C), (1, 4 * C), (C, 4 * C), (1, C)]
    cls_out, patch_out = _call(_kc_body, batch, kc_ins, kc_shapes,
                               [(CLSN, C), (C, N)])
    return cls_out, patch_out.reshape(batch, C, HP, WP)
